# baseline (speedup 1.0000x reference)
import functools

import jax
import jax.numpy as jnp
from jax import lax
from jax.experimental import pallas as pl
from jax.experimental.pallas import tpu as pltpu
from jax.experimental.pallas import tpu_sc as plsc

DIM = 64
SEQ = 128
EPS = 1e-5
NC = 2
NS = 16
NW = NC * NS
CHUNK = 128


def _rsqrt(x):
    i = plsc.bitcast(x, jnp.int32)
    i = jnp.int32(0x5F3759DF) - lax.shift_right_logical(i, 1)
    y = plsc.bitcast(i, jnp.float32)
    for _ in range(3):
        y = y * (1.5 - 0.5 * x * y * y)
    return y


def _lane_sum(v):
    idx = lax.iota(jnp.int32, 16)
    dnums = lax.GatherDimensionNumbers(
        offset_dims=(), collapsed_slice_dims=(0,), start_index_map=(0,))
    for k in (8, 4, 2, 1):
        perm = lax.bitwise_xor(idx, jnp.int32(k))
        v = v + lax.gather(v, perm[:, None], dnums, slice_sizes=(1,),
                           mode=lax.GatherScatterMode.PROMISE_IN_BOUNDS)
    return v


def _sc_body(total_rows, ids_hbm, table_hbm, pos_hbm, gamma_hbm, beta_hbm,
             out_hbm, idx_v, rows_v, pos_v, gamma_v, beta_v, sem):
    wid = lax.axis_index("s") * NC + lax.axis_index("c")
    rows_per_w = total_rows // NW
    base = pl.multiple_of(wid * rows_per_w, CHUNK)

    pltpu.sync_copy(ids_hbm.at[pl.ds(base, rows_per_w)], idx_v)
    pltpu.sync_copy(pos_hbm, pos_v)
    pltpu.sync_copy(gamma_hbm, gamma_v)
    pltpu.sync_copy(beta_hbm, beta_v)

    g = [gamma_v[pl.ds(16 * j, 16)] for j in range(4)]
    b = [beta_v[pl.ds(16 * j, 16)] for j in range(4)]
    inv_d = jnp.float32(1.0 / DIM)

    nchunks = rows_per_w // CHUNK

    def chunk_body(c, _):
        off = pl.multiple_of(c * CHUNK, CHUNK)
        pltpu.async_copy(
            table_hbm.at[idx_v.at[pl.ds(off, CHUNK)]], rows_v, sem
        ).wait()

        def row_body(r, _):
            x = [rows_v[r, pl.ds(16 * j, 16)] + pos_v[r, pl.ds(16 * j, 16)]
                 for j in range(4)]
            s = _lane_sum(x[0] + x[1] + x[2] + x[3])
            q = _lane_sum(x[0] * x[0] + x[1] * x[1]
                          + x[2] * x[2] + x[3] * x[3])
            mean = s * inv_d
            var = q * inv_d - mean * mean
            rstd = _rsqrt(var + EPS)
            for j in range(4):
                rows_v[r, pl.ds(16 * j, 16)] = (x[j] - mean) * rstd * g[j] + b[j]
            return 0

        lax.fori_loop(0, CHUNK, row_body, 0)
        pltpu.sync_copy(rows_v, out_hbm.at[pl.ds(base + off, CHUNK)])
        return 0

    lax.fori_loop(0, nchunks, chunk_body, 0)


def kernel(input_ids, token_table, pos_table, gamma, beta):
    batch, seq = input_ids.shape
    total_rows = batch * seq
    ids_flat = input_ids.reshape(total_rows).astype(jnp.int32)
    rows_per_w = total_rows // NW

    mesh = plsc.VectorSubcoreMesh(core_axis_name="c", subcore_axis_name="s")
    out = pl.kernel(
        functools.partial(_sc_body, total_rows),
        out_type=jax.ShapeDtypeStruct((total_rows, DIM), jnp.float32),
        mesh=mesh,
        compiler_params=pltpu.CompilerParams(
            needs_layout_passes=False, use_tc_tiling_on_sc=False),
        scratch_types=[
            pltpu.VMEM((rows_per_w,), jnp.int32),
            pltpu.VMEM((CHUNK, DIM), jnp.float32),
            pltpu.VMEM((SEQ, DIM), jnp.float32),
            pltpu.VMEM((DIM,), jnp.float32),
            pltpu.VMEM((DIM,), jnp.float32),
            pltpu.SemaphoreType.DMA,
        ],
    )(ids_flat, token_table, pos_table, gamma, beta)
    return out.reshape(batch, seq, DIM)


# speedup vs baseline: 1.5719x; 1.5719x over previous
import functools

import jax
import jax.numpy as jnp
from jax import lax
from jax.experimental import pallas as pl
from jax.experimental.pallas import tpu as pltpu
from jax.experimental.pallas import tpu_sc as plsc

DIM = 64
SEQ = 128
EPS = 1e-5
NC = 2
NS = 16
NW = NC * NS
CHUNK = 128
NBUF = 4


def _rsqrt(x):
    i = plsc.bitcast(x, jnp.int32)
    i = jnp.int32(0x5F3759DF) - lax.shift_right_logical(i, 1)
    y = plsc.bitcast(i, jnp.float32)
    hx = 0.5 * x
    for _ in range(2):
        y = y * (1.5 - hx * y * y)
    return y


def _lane_sum(v):
    idx = lax.iota(jnp.int32, 16)
    dnums = lax.GatherDimensionNumbers(
        offset_dims=(), collapsed_slice_dims=(0,), start_index_map=(0,))
    for k in (8, 4, 2, 1):
        perm = lax.bitwise_xor(idx, jnp.int32(k))
        v = v + lax.gather(v, perm[:, None], dnums, slice_sizes=(1,),
                           mode=lax.GatherScatterMode.PROMISE_IN_BOUNDS)
    return v


def _sc_body(total_rows, ids_hbm, table_hbm, pos_hbm, gamma_hbm, beta_hbm,
             out_hbm, idx_v, rows_v, pos_v, gamma_v, beta_v, in_sems,
             out_sems):
    wid = lax.axis_index("s") * NC + lax.axis_index("c")
    rows_per_w = total_rows // NW
    base = pl.multiple_of(wid * rows_per_w, CHUNK)

    pltpu.sync_copy(ids_hbm.at[pl.ds(base, rows_per_w)], idx_v)
    pltpu.sync_copy(pos_hbm, pos_v)
    pltpu.sync_copy(gamma_hbm, gamma_v)
    pltpu.sync_copy(beta_hbm, beta_v)

    g = [gamma_v[pl.ds(16 * j, 16)] for j in range(4)]
    b = [beta_v[pl.ds(16 * j, 16)] for j in range(4)]
    inv_d = jnp.float32(1.0 / DIM)

    nchunks = rows_per_w // CHUNK

    def start_gather(c, buf):
        off = pl.multiple_of(c * CHUNK, CHUNK)
        pltpu.async_copy(table_hbm.at[idx_v.at[pl.ds(off, CHUNK)]],
                         rows_v.at[buf], in_sems.at[buf])

    def wait_dma(dst, sem):
        pltpu.make_async_copy(out_hbm.at[pl.ds(0, CHUNK)], dst, sem).wait()

    start_gather(0, 0)
    start_gather(1, 1)

    def group_body(grp, _):
        for bi in range(NBUF):
            c = grp * NBUF + bi
            buf = rows_v.at[bi]
            wait_dma(buf, in_sems.at[bi])

            @plsc.parallel_loop(0, CHUNK, unroll=4)
            def row_body(r):
                x = [buf[r, pl.ds(16 * j, 16)] + pos_v[r, pl.ds(16 * j, 16)]
                     for j in range(4)]
                s = _lane_sum(x[0] + x[1] + x[2] + x[3])
                q = _lane_sum(x[0] * x[0] + x[1] * x[1]
                              + x[2] * x[2] + x[3] * x[3])
                mean = s * inv_d
                var = q * inv_d - mean * mean
                rstd = _rsqrt(var + EPS)
                for j in range(4):
                    buf[r, pl.ds(16 * j, 16)] = \
                        (x[j] - mean) * rstd * g[j] + b[j]

            off = pl.multiple_of(c * CHUNK, CHUNK)
            pltpu.async_copy(buf, out_hbm.at[pl.ds(base + off, CHUNK)],
                             out_sems.at[bi])

            nxt = c + 2
            nbi = (bi + 2) % NBUF

            @pl.when(nxt < nchunks)
            def _():
                @pl.when(nxt >= NBUF)
                def _():
                    wait_dma(rows_v.at[nbi], out_sems.at[nbi])
                start_gather(nxt, nbi)

        return 0

    lax.fori_loop(0, nchunks // NBUF, group_body, 0)

    for bi in range(NBUF):
        wait_dma(rows_v.at[bi], out_sems.at[bi])


def kernel(input_ids, token_table, pos_table, gamma, beta):
    batch, seq = input_ids.shape
    total_rows = batch * seq
    ids_flat = input_ids.reshape(total_rows).astype(jnp.int32)
    rows_per_w = total_rows // NW

    mesh = plsc.VectorSubcoreMesh(core_axis_name="c", subcore_axis_name="s")
    out = pl.kernel(
        functools.partial(_sc_body, total_rows),
        out_type=jax.ShapeDtypeStruct((total_rows, DIM), jnp.float32),
        mesh=mesh,
        compiler_params=pltpu.CompilerParams(
            needs_layout_passes=False, use_tc_tiling_on_sc=False),
        scratch_types=[
            pltpu.VMEM((rows_per_w,), jnp.int32),
            pltpu.VMEM((NBUF, CHUNK, DIM), jnp.float32),
            pltpu.VMEM((SEQ, DIM), jnp.float32),
            pltpu.VMEM((DIM,), jnp.float32),
            pltpu.VMEM((DIM,), jnp.float32),
            pltpu.SemaphoreType.DMA((NBUF,)),
            pltpu.SemaphoreType.DMA((NBUF,)),
        ],
    )(ids_flat, token_table, pos_table, gamma, beta)
    return out.reshape(batch, seq, DIM)
